# baseline (speedup 1.0000x reference)
import functools

import jax
import jax.numpy as jnp
from jax import lax
from jax.experimental import pallas as pl
from jax.experimental.pallas import tpu as pltpu
from jax.experimental.pallas import tpu_sc as plsc

DIM = 64
LANES = 16
NC, NS = 2, 16
NW = NC * NS
C = 128

def _rsqrt(x):
    i = plsc.bitcast(x, jnp.int32)
    i = jnp.full((LANES,), 0x5F3759DF, jnp.int32) - lax.shift_right_logical(i, 1)
    y = plsc.bitcast(i, jnp.float32)
    for _ in range(3):
        y = y * (1.5 - 0.5 * x * y * y)
    return y


def _score_pass(h_rows, t_rows, r_rows, out_ref, out_base):
    def group_body(g, carry):
        rows = g * LANES + lax.iota(jnp.int32, LANES)
        zero = jnp.zeros((LANES,), jnp.float32)
        hh = zero; tt = zero; rr = zero
        hr = zero; ht = zero; rt = zero
        for d in range(DIM):
            col = jnp.full((LANES,), d, jnp.int32)
            hv = plsc.load_gather(h_rows, [rows, col])
            tv = plsc.load_gather(t_rows, [rows, col])
            rv = plsc.load_gather(r_rows, [rows, col])
            hh = hh + hv * hv
            tt = tt + tv * tv
            rr = rr + rv * rv
            hr = hr + hv * rv
            ht = ht + hv * tv
            rt = rt + rv * tv
        ih = _rsqrt(jnp.maximum(hh, 1e-24))
        it = _rsqrt(jnp.maximum(tt, 1e-24))
        g2 = ((hh * ih) * ih + rr + (tt * it) * it
              + 2.0 * ((hr * ih) - (ht * ih) * it - (rt * it)))
        g2 = jnp.maximum(g2, 0.0)
        res = g2 * _rsqrt(jnp.maximum(g2, 1e-24))
        out_ref[pl.ds(out_base + g * LANES, LANES)] = -res
        return carry

    lax.fori_loop(0, C // LANES, group_body, jnp.int32(0))


def _make_sc_call(batch):
    assert batch % NW == 0
    pw = batch // NW
    assert pw % C == 0
    n_chunks = pw // C
    mesh = plsc.VectorSubcoreMesh(core_axis_name="c", subcore_axis_name="s")

    @functools.partial(
        pl.kernel,
        mesh=mesh,
        compiler_params=pltpu.CompilerParams(
            use_tc_tiling_on_sc=False, needs_layout_passes=False),
        out_type=(
            jax.ShapeDtypeStruct((batch,), jnp.float32),
            jax.ShapeDtypeStruct((batch,), jnp.float32),
        ),
        scratch_types=[
            pltpu.VMEM((pw,), jnp.int32),
            pltpu.VMEM((pw,), jnp.int32),
            pltpu.VMEM((pw,), jnp.int32),
            pltpu.VMEM((pw,), jnp.int32),
            pltpu.VMEM((pw,), jnp.int32),
            pltpu.VMEM((C, DIM), jnp.float32),
            pltpu.VMEM((C, DIM), jnp.float32),
            pltpu.VMEM((C, DIM), jnp.float32),
            pltpu.VMEM((C, DIM), jnp.float32),
            pltpu.VMEM((C, DIM), jnp.float32),
            pltpu.VMEM((pw,), jnp.float32),
            pltpu.VMEM((pw,), jnp.float32),
            pltpu.SemaphoreType.DMA,
            pltpu.SemaphoreType.DMA,
        ],
    )
    def sc_call(heads, tails, nheads, ntails, rels, ent, rel_emb,
                out_g, out_n,
                hidx, tidx, nhidx, ntidx, ridx,
                h_rows, t_rows, r_rows, nh_rows, nt_rows,
                og, on, sem_a, sem_b):
        wid = lax.axis_index("s") * NC + lax.axis_index("c")
        base = pl.multiple_of(wid * pw, pw)
        pltpu.sync_copy(heads.at[pl.ds(base, pw)], hidx)
        pltpu.sync_copy(tails.at[pl.ds(base, pw)], tidx)
        pltpu.sync_copy(nheads.at[pl.ds(base, pw)], nhidx)
        pltpu.sync_copy(ntails.at[pl.ds(base, pw)], ntidx)
        pltpu.sync_copy(rels.at[pl.ds(base, pw)], ridx)

        def chunk_body(ch, carry):
            off = pl.multiple_of(ch * C, C)
            a1 = pltpu.async_copy(ent.at[hidx.at[pl.ds(off, C)]], h_rows, sem_a)
            a2 = pltpu.async_copy(ent.at[tidx.at[pl.ds(off, C)]], t_rows, sem_a)
            a3 = pltpu.async_copy(rel_emb.at[ridx.at[pl.ds(off, C)]], r_rows, sem_a)
            a4 = pltpu.async_copy(ent.at[nhidx.at[pl.ds(off, C)]], nh_rows, sem_b)
            a5 = pltpu.async_copy(ent.at[ntidx.at[pl.ds(off, C)]], nt_rows, sem_b)
            a1.wait()
            a2.wait()
            a3.wait()
            _score_pass(h_rows, t_rows, r_rows, og, off)
            a4.wait()
            a5.wait()
            _score_pass(nh_rows, nt_rows, r_rows, on, off)
            return carry

        lax.fori_loop(0, n_chunks, chunk_body, jnp.int32(0))
        pltpu.sync_copy(og, out_g.at[pl.ds(base, pw)])
        pltpu.sync_copy(on, out_n.at[pl.ds(base, pw)])

    return sc_call


def kernel(heads, tails, negative_heads, negative_tails, relations,
           entity_embeddings, relation_embeddings):
    batch = heads.shape[0]
    sc_call = _make_sc_call(batch)
    return sc_call(heads, tails, negative_heads, negative_tails, relations,
                   entity_embeddings, relation_embeddings)


# speedup vs baseline: 1.3045x; 1.3045x over previous
import functools

import jax
import jax.numpy as jnp
from jax import lax
from jax.experimental import pallas as pl
from jax.experimental.pallas import tpu as pltpu
from jax.experimental.pallas import tpu_sc as plsc

B = 16384
N_ENT = 1000000
DIM = 64
L = 16
NC, NS = 2, 16
NW = NC * NS
COLS = (N_ENT + 127) // 128
CPW = 256
PAIRS = 4 * B
SELCAP = 8192
BRIDGE_ROWS = PAIRS + NW * L
PW = B // NW

_PARAMS = pltpu.CompilerParams(needs_layout_passes=False)


def _rsqrt(x):
    i = plsc.bitcast(x, jnp.int32)
    i = jnp.full((L,), 0x5F3759DF, jnp.int32) - lax.shift_right_logical(i, 1)
    y = plsc.bitcast(i, jnp.float32)
    for _ in range(3):
        y = y * (1.5 - 0.5 * x * y * y)
    return y


def _iota():
    return lax.iota(jnp.int32, L)


def _scalar(v16):
    return v16[0]


def _lane0():
    return lax.iota(jnp.int32, L) == 0


def _sload(ref, i):
    return plsc.load_gather(ref, [jnp.full((L,), i, jnp.int32)])[0]


def _sstore(ref, i, val, lane0, dtype=jnp.int32):
    plsc.store_scatter(ref, [jnp.full((L,), i, jnp.int32)],
                       jnp.full((L,), val, dtype), mask=lane0)


def _make_k1():
    mesh = plsc.VectorSubcoreMesh(core_axis_name="c", subcore_axis_name="s")

    @functools.partial(
        pl.kernel,
        mesh=mesh,
        compiler_params=_PARAMS,
        out_type=jax.ShapeDtypeStruct((BRIDGE_ROWS, 128), jnp.float32),
        scratch_types=[
            pltpu.VMEM((8192,), jnp.int32),
            pltpu.VMEM((SELCAP + 16,), jnp.int32),
            pltpu.VMEM((SELCAP + 16,), jnp.int32),
            pltpu.VMEM((SELCAP,), jnp.int32),
            pltpu.VMEM((SELCAP,), jnp.int32),
            pltpu.VMEM((272,), jnp.int32),
            pltpu.VMEM((272,), jnp.int32),
            pltpu.VMEM((64, 128), jnp.float32),
            pltpu.VMEM((64, 128), jnp.float32),
            pltpu.VMEM((128, 128), jnp.float32),
            pltpu.VMEM((128,), jnp.int32),
            pltpu.SemaphoreType.DMA,
            pltpu.SemaphoreType.DMA,
            pltpu.SemaphoreType.DMA,
        ],
    )
    def k1(heads, tails, nheads, ntails, tt, bridge,
           ibuf, sel_e, sel_d, srt_e, srt_d, bounds, cur,
           colbuf0, colbuf1, rowbuf, destbuf, sem0, sem1, sem_s):
        w = lax.axis_index("s") * NC + lax.axis_index("c")
        lo = w * (CPW * 128)
        hi = lo + CPW * 128
        iota = _iota()

        n_sel = jnp.int32(0)
        for role, arr in enumerate((heads, tails, nheads, ntails)):
            for blk in range(2):
                pltpu.sync_copy(arr.at[pl.ds(blk * 8192, 8192)], ibuf)
                dbase = role * B + blk * 8192

                def scan_body(c, n, dbase=dbase):
                    v = ibuf[pl.ds(c * L, L)]
                    m = (v >= lo) & (v < hi)
                    plsc.store_compressed(sel_e.at[pl.ds(n, L)], v, mask=m)
                    d = dbase + c * L + iota
                    plsc.store_compressed(sel_d.at[pl.ds(n, L)], d, mask=m)
                    return n + _scalar(plsc.all_reduce_population_count(m))

                n_sel = lax.fori_loop(0, 8192 // L, scan_body, n_sel)

        for k in range(272 // L):
            cur[pl.ds(k * L, L)] = jnp.zeros((L,), jnp.int32)

        lane0 = _lane0()
        ones = jnp.full((L,), 1, jnp.int32)

        def hist_body(i, _):
            c = (_sload(sel_e, i) >> 7) - w * CPW
            plsc.addupdate_scatter(cur, [jnp.full((L,), c, jnp.int32)], ones,
                                   mask=lane0)
            return _

        lax.fori_loop(0, n_sel, hist_body, jnp.int32(0))

        def prefix_body(c, run):
            t = _sload(cur, c)
            _sstore(bounds, c, run, lane0)
            _sstore(cur, c, run, lane0)
            return run + t

        total = lax.fori_loop(0, 256, prefix_body, jnp.int32(0))
        _sstore(bounds, jnp.int32(256), total, lane0)

        def place_body(i, _):
            e = _sload(sel_e, i)
            c = (e >> 7) - w * CPW
            p = _sload(cur, c)
            _sstore(cur, c, p + 1, lane0)
            _sstore(srt_e, p, e, lane0)
            _sstore(srt_d, p, _sload(sel_d, i), lane0)
            return _

        lax.fori_loop(0, n_sel, place_body, jnp.int32(0))

        ncols = jnp.minimum(jnp.int32(CPW), jnp.maximum(jnp.int32(COLS) - w * CPW,
                                                        jnp.int32(0)))
        for k in range(128 // L):
            destbuf[pl.ds(k * L, L)] = PAIRS + w * L + iota

        def issue(j, cb, sem):
            basej = (w * CPW + j) * 128
            pltpu.async_copy(tt.at[:, pl.ds(basej, 128)], cb, sem)

        def drain(j, cb, sem):
            basej = (w * CPW + j) * 128
            pltpu.make_async_copy(tt.at[:, pl.ds(basej, 128)], cb, sem).wait()

        @pl.when(ncols > 0)
        def _():
            issue(jnp.int32(0), colbuf0, sem0)

        @pl.when(ncols > 1)
        def _():
            issue(jnp.int32(1), colbuf1, sem1)

        rows16 = [iota + 16 * k for k in range(4)]

        def serve_col(j, nfill, cb, sem):
            drain(j, cb, sem)
            base = (w * CPW + j) * 128

            lane0 = _lane0()

            def pair_body(p, nf):
                e = _sload(srt_e, p)
                el = e - base
                colv = jnp.full((L,), el, jnp.int32)
                nfv = jnp.full((L,), nf, jnp.int32)
                for k in range(4):
                    v = plsc.load_gather(cb, [rows16[k], colv])
                    plsc.store_scatter(rowbuf, [nfv, rows16[k]], v)
                _sstore(destbuf, nf, _sload(srt_d, p), lane0)
                nf = nf + 1

                @pl.when(nf == 128)
                def _():
                    pltpu.async_copy(rowbuf, bridge.at[destbuf], sem_s).wait()

                return lax.select(nf == 128, jnp.int32(0), nf)

            nfill = lax.fori_loop(_sload(bounds, j), _sload(bounds, j + 1),
                                  pair_body, nfill)

            @pl.when(j + 2 < ncols)
            def _():
                issue(j + 2, cb, sem)

            return nfill

        def col_body(j, nfill):
            nf0 = lax.cond(j % 2 == 0,
                           lambda nf: serve_col(j, nf, colbuf0, sem0),
                           lambda nf: serve_col(j, nf, colbuf1, sem1),
                           nfill)
            return nf0

        nfill = lax.fori_loop(0, ncols, col_body, jnp.int32(0))

        @pl.when(nfill > 0)
        def _():
            pltpu.async_copy(rowbuf, bridge.at[destbuf], sem_s).wait()

    return k1


def _make_k2():
    mesh = plsc.VectorSubcoreMesh(core_axis_name="c", subcore_axis_name="s")

    @functools.partial(
        pl.kernel,
        mesh=mesh,
        compiler_params=_PARAMS,
        out_type=(
            jax.ShapeDtypeStruct((B,), jnp.float32),
            jax.ShapeDtypeStruct((B,), jnp.float32),
        ),
        scratch_types=[
            pltpu.VMEM((500, 128), jnp.float32),
            pltpu.VMEM((PW,), jnp.int32),
            pltpu.VMEM((64, 128), jnp.float32),
            pltpu.VMEM((64, 128), jnp.float32),
            pltpu.VMEM((64, 128), jnp.float32),
            pltpu.VMEM((64, 128), jnp.float32),
            pltpu.VMEM((PW,), jnp.float32),
            pltpu.VMEM((PW,), jnp.float32),
            pltpu.SemaphoreType.DMA,
        ],
    )
    def k2(bridge, relp, relations, out_g, out_n,
           rel_v, ridx, h_v, t_v, nh_v, nt_v, og, on, sem):
        w = lax.axis_index("s") * NC + lax.axis_index("c")
        base = pl.multiple_of(w * PW, PW)
        pltpu.sync_copy(relp, rel_v)
        pltpu.sync_copy(relations.at[pl.ds(base, PW)], ridx)
        iota = _iota()

        def score_pass(a_v, b_v, off, out_ref):
            for g in range(4):
                r16 = iota + g * L
                q = ridx[pl.ds(off + g * L, L)]
                qrow = lax.shift_right_logical(q, 1)
                qcol0 = (q & 1) * 64
                zero = jnp.zeros((L,), jnp.float32)
                hh = zero; tt_ = zero; rr = zero
                hr = zero; ht = zero; rt = zero
                for d in range(DIM):
                    dv = jnp.full((L,), d, jnp.int32)
                    hv = plsc.load_gather(a_v, [r16, dv])
                    tv = plsc.load_gather(b_v, [r16, dv])
                    rv = plsc.load_gather(rel_v, [qrow, qcol0 + dv])
                    hh = hh + hv * hv
                    tt_ = tt_ + tv * tv
                    rr = rr + rv * rv
                    hr = hr + hv * rv
                    ht = ht + hv * tv
                    rt = rt + rv * tv
                ih = _rsqrt(jnp.maximum(hh, 1e-24))
                it = _rsqrt(jnp.maximum(tt_, 1e-24))
                g2 = ((hh * ih) * ih + rr + (tt_ * it) * it
                      + 2.0 * ((hr * ih) - (ht * ih) * it - (rt * it)))
                g2 = jnp.maximum(g2, 0.0)
                res = g2 * _rsqrt(jnp.maximum(g2, 1e-24))
                out_ref[pl.ds(off + g * L, L)] = -res

        def chunk_body(c, carry):
            slot0 = base + c * 64
            a1 = pltpu.async_copy(bridge.at[pl.ds(slot0, 64)], h_v, sem)
            a2 = pltpu.async_copy(bridge.at[pl.ds(B + slot0, 64)], t_v, sem)
            a3 = pltpu.async_copy(bridge.at[pl.ds(2 * B + slot0, 64)], nh_v, sem)
            a4 = pltpu.async_copy(bridge.at[pl.ds(3 * B + slot0, 64)], nt_v, sem)
            a1.wait(); a2.wait(); a3.wait(); a4.wait()
            score_pass(h_v, t_v, c * 64, og)
            score_pass(nh_v, nt_v, c * 64, on)
            return carry

        lax.fori_loop(0, PW // 64, chunk_body, jnp.int32(0))
        pltpu.sync_copy(og, out_g.at[pl.ds(base, PW)])
        pltpu.sync_copy(on, out_n.at[pl.ds(base, PW)])

    return k2


def kernel(heads, tails, negative_heads, negative_tails, relations,
           entity_embeddings, relation_embeddings):
    tt = entity_embeddings.T
    relp = relation_embeddings.reshape(500, 128)
    bridge = _make_k1()(heads, tails, negative_heads, negative_tails, tt)
    return _make_k2()(bridge, relp, relations)


# speedup vs baseline: 1.4491x; 1.1109x over previous
import functools

import jax
import jax.numpy as jnp
from jax import lax
from jax.experimental import pallas as pl
from jax.experimental.pallas import tpu as pltpu
from jax.experimental.pallas import tpu_sc as plsc

B = 16384
N_ENT = 1000000
DIM = 64
L = 16
NC, NS = 2, 16
NW = NC * NS
COLS = (N_ENT + 127) // 128
CPW = 256
PAIRS = 4 * B
SELCAP = 8192
BRIDGE_ROWS = PAIRS + NW * L
PW = B // NW

_PARAMS = pltpu.CompilerParams(needs_layout_passes=False)


def _rsqrt(x):
    i = plsc.bitcast(x, jnp.int32)
    i = jnp.full((L,), 0x5F3759DF, jnp.int32) - lax.shift_right_logical(i, 1)
    y = plsc.bitcast(i, jnp.float32)
    for _ in range(3):
        y = y * (1.5 - 0.5 * x * y * y)
    return y


def _iota():
    return lax.iota(jnp.int32, L)


def _scalar(v16):
    return v16[0]


def _lane0():
    return lax.iota(jnp.int32, L) == 0


def _sload(ref, i):
    return plsc.load_gather(ref, [jnp.full((L,), i, jnp.int32)])[0]


def _sstore(ref, i, val, lane0, dtype=jnp.int32):
    plsc.store_scatter(ref, [jnp.full((L,), i, jnp.int32)],
                       jnp.full((L,), val, dtype), mask=lane0)


def _make_k1():
    mesh = plsc.VectorSubcoreMesh(core_axis_name="c", subcore_axis_name="s")

    @functools.partial(
        pl.kernel,
        mesh=mesh,
        compiler_params=_PARAMS,
        out_type=jax.ShapeDtypeStruct((BRIDGE_ROWS, 128), jnp.float32),
        scratch_types=[
            pltpu.VMEM((8192,), jnp.int32),
            pltpu.VMEM((SELCAP + 16,), jnp.int32),
            pltpu.VMEM((SELCAP + 16,), jnp.int32),
            pltpu.VMEM((SELCAP + 16,), jnp.int32),
            pltpu.VMEM((SELCAP + 16,), jnp.int32),
            pltpu.VMEM((272,), jnp.int32),
            pltpu.VMEM((272,), jnp.int32),
            pltpu.VMEM((64, 128), jnp.float32),
            pltpu.VMEM((64, 128), jnp.float32),
            pltpu.VMEM((128, 128), jnp.float32),
            pltpu.VMEM((128,), jnp.int32),
            pltpu.SemaphoreType.DMA,
            pltpu.SemaphoreType.DMA,
            pltpu.SemaphoreType.DMA,
        ],
    )
    def k1(heads, tails, nheads, ntails, tt, bridge,
           ibuf, sel_e, sel_d, srt_e, srt_d, bounds, cur,
           colbuf0, colbuf1, rowbuf, destbuf, sem0, sem1, sem_s):
        w = lax.axis_index("s") * NC + lax.axis_index("c")
        lo = w * (CPW * 128)
        hi = lo + CPW * 128
        iota = _iota()

        n_sel = jnp.int32(0)
        for role, arr in enumerate((heads, tails, nheads, ntails)):
            for blk in range(2):
                pltpu.sync_copy(arr.at[pl.ds(blk * 8192, 8192)], ibuf)
                dbase = role * B + blk * 8192

                def scan_body(c, n, dbase=dbase):
                    v = ibuf[pl.ds(c * L, L)]
                    m = (v >= lo) & (v < hi)
                    plsc.store_compressed(sel_e.at[pl.ds(n, L)], v, mask=m)
                    d = dbase + c * L + iota
                    plsc.store_compressed(sel_d.at[pl.ds(n, L)], d, mask=m)
                    return n + _scalar(plsc.all_reduce_population_count(m))

                n_sel = lax.fori_loop(0, 8192 // L, scan_body, n_sel)

        for k in range(272 // L):
            cur[pl.ds(k * L, L)] = jnp.zeros((L,), jnp.int32)

        lane0 = _lane0()
        ones = jnp.full((L,), 1, jnp.int32)
        nvec = lax.shift_right_logical(n_sel + (L - 1), 4)

        def hist_body(k, _):
            i0 = k * L
            valid = (i0 + iota) < n_sel
            e16 = sel_e[pl.ds(i0, L)]
            c16 = jnp.where(valid, ((e16 >> 7) - w * CPW) & 255, 256 + iota)
            plsc.addupdate_scatter(cur, [c16], ones)
            return _

        lax.fori_loop(0, nvec, hist_body, jnp.int32(0))

        def prefix_body(k, run_v):
            v = cur[pl.ds(k * L, L)]
            s = v
            for step in (1, 2, 4, 8):
                ibuf[pl.ds(0, L)] = s
                sh = plsc.load_gather(ibuf, [jnp.maximum(iota - step, 0)])
                s = s + jnp.where(iota >= step, sh, 0)
            excl = (s - v) + run_v
            bounds[pl.ds(k * L, L)] = excl
            cur[pl.ds(k * L, L)] = excl
            ibuf[pl.ds(0, L)] = s
            tot = plsc.load_gather(ibuf, [jnp.full((L,), 15, jnp.int32)])
            return run_v + tot

        run_v = lax.fori_loop(0, 256 // L, prefix_body,
                              jnp.zeros((L,), jnp.int32))
        plsc.store_scatter(bounds, [jnp.full((L,), 256, jnp.int32)], run_v,
                           mask=lane0)

        def place_body(k, _):
            i0 = k * L
            valid = (i0 + iota) < n_sel
            e16 = sel_e[pl.ds(i0, L)]
            d16 = sel_d[pl.ds(i0, L)]
            c16 = jnp.where(valid, ((e16 >> 7) - w * CPW) & 255, 256 + iota)
            ibuf[pl.ds(0, L)] = c16
            rank = jnp.zeros((L,), jnp.int32)
            for s in range(1, L):
                sh = plsc.load_gather(ibuf, [jnp.maximum(iota - s, 0)])
                rank = rank + jnp.where((sh == c16) & (iota >= s), 1, 0)
            p16 = plsc.load_gather(cur, [c16]) + rank
            p16 = jnp.where(valid, jnp.clip(p16, 0, SELCAP - 1), SELCAP + iota)
            plsc.store_scatter(srt_e, [p16], e16)
            plsc.store_scatter(srt_d, [p16], d16)
            plsc.store_scatter(cur, [c16], p16 + 1)
            return _

        lax.fori_loop(0, nvec, place_body, jnp.int32(0))

        ncols = jnp.minimum(jnp.int32(CPW), jnp.maximum(jnp.int32(COLS) - w * CPW,
                                                        jnp.int32(0)))
        for k in range(128 // L):
            destbuf[pl.ds(k * L, L)] = PAIRS + w * L + iota

        def issue(j, cb, sem):
            basej = (w * CPW + j) * 128
            pltpu.async_copy(tt.at[:, pl.ds(basej, 128)], cb, sem)

        def drain(j, cb, sem):
            basej = (w * CPW + j) * 128
            pltpu.make_async_copy(tt.at[:, pl.ds(basej, 128)], cb, sem).wait()

        @pl.when(ncols > 0)
        def _():
            issue(jnp.int32(0), colbuf0, sem0)

        @pl.when(ncols > 1)
        def _():
            issue(jnp.int32(1), colbuf1, sem1)

        rows16 = [iota + 16 * k for k in range(4)]

        def serve_col(j, nfill, cb, sem):
            drain(j, cb, sem)
            base = (w * CPW + j) * 128

            lane0 = _lane0()

            def pair_body(p, nf):
                e = _sload(srt_e, p)
                el = (e - base) & 127
                colv = jnp.full((L,), el, jnp.int32)
                nfv = jnp.full((L,), nf, jnp.int32)
                for k in range(4):
                    v = plsc.load_gather(cb, [rows16[k], colv])
                    plsc.store_scatter(rowbuf, [nfv, rows16[k]], v)
                dest = jnp.clip(_sload(srt_d, p), 0, BRIDGE_ROWS - 1)
                _sstore(destbuf, nf, dest, lane0)
                nf = nf + 1

                @pl.when(nf == 128)
                def _():
                    pltpu.async_copy(rowbuf, bridge.at[destbuf], sem_s).wait()

                return lax.select(nf == 128, jnp.int32(0), nf)

            lo_b = jnp.clip(_sload(bounds, j), 0, n_sel)
            hi_b = jnp.clip(_sload(bounds, j + 1), lo_b, n_sel)
            nfill = lax.fori_loop(lo_b, hi_b, pair_body, nfill)

            @pl.when(j + 2 < ncols)
            def _():
                issue(j + 2, cb, sem)

            return nfill

        def col_body(j, nfill):
            nf0 = lax.cond(j % 2 == 0,
                           lambda nf: serve_col(j, nf, colbuf0, sem0),
                           lambda nf: serve_col(j, nf, colbuf1, sem1),
                           nfill)
            return nf0

        nfill = lax.fori_loop(0, ncols, col_body, jnp.int32(0))

        @pl.when(nfill > 0)
        def _():
            pltpu.async_copy(rowbuf, bridge.at[destbuf], sem_s).wait()

    return k1


def _make_k2():
    mesh = plsc.VectorSubcoreMesh(core_axis_name="c", subcore_axis_name="s")

    @functools.partial(
        pl.kernel,
        mesh=mesh,
        compiler_params=_PARAMS,
        out_type=(
            jax.ShapeDtypeStruct((B,), jnp.float32),
            jax.ShapeDtypeStruct((B,), jnp.float32),
        ),
        scratch_types=[
            pltpu.VMEM((500, 128), jnp.float32),
            pltpu.VMEM((PW,), jnp.int32),
            pltpu.VMEM((64, 128), jnp.float32),
            pltpu.VMEM((64, 128), jnp.float32),
            pltpu.VMEM((64, 128), jnp.float32),
            pltpu.VMEM((64, 128), jnp.float32),
            pltpu.VMEM((PW,), jnp.float32),
            pltpu.VMEM((PW,), jnp.float32),
            pltpu.SemaphoreType.DMA,
        ],
    )
    def k2(bridge, relp, relations, out_g, out_n,
           rel_v, ridx, h_v, t_v, nh_v, nt_v, og, on, sem):
        w = lax.axis_index("s") * NC + lax.axis_index("c")
        base = pl.multiple_of(w * PW, PW)
        pltpu.sync_copy(relp, rel_v)
        pltpu.sync_copy(relations.at[pl.ds(base, PW)], ridx)
        iota = _iota()

        def score_pass(a_v, b_v, off, out_ref):
            for g in range(4):
                r16 = iota + g * L
                q = ridx[pl.ds(off + g * L, L)]
                qrow = lax.shift_right_logical(q, 1)
                qcol0 = (q & 1) * 64
                zero = jnp.zeros((L,), jnp.float32)
                hh = zero; tt_ = zero; rr = zero
                hr = zero; ht = zero; rt = zero
                for d in range(DIM):
                    dv = jnp.full((L,), d, jnp.int32)
                    hv = plsc.load_gather(a_v, [r16, dv])
                    tv = plsc.load_gather(b_v, [r16, dv])
                    rv = plsc.load_gather(rel_v, [qrow, qcol0 + dv])
                    hh = hh + hv * hv
                    tt_ = tt_ + tv * tv
                    rr = rr + rv * rv
                    hr = hr + hv * rv
                    ht = ht + hv * tv
                    rt = rt + rv * tv
                ih = _rsqrt(jnp.maximum(hh, 1e-24))
                it = _rsqrt(jnp.maximum(tt_, 1e-24))
                g2 = ((hh * ih) * ih + rr + (tt_ * it) * it
                      + 2.0 * ((hr * ih) - (ht * ih) * it - (rt * it)))
                g2 = jnp.maximum(g2, 0.0)
                res = g2 * _rsqrt(jnp.maximum(g2, 1e-24))
                out_ref[pl.ds(off + g * L, L)] = -res

        def chunk_body(c, carry):
            slot0 = base + c * 64
            a1 = pltpu.async_copy(bridge.at[pl.ds(slot0, 64)], h_v, sem)
            a2 = pltpu.async_copy(bridge.at[pl.ds(B + slot0, 64)], t_v, sem)
            a3 = pltpu.async_copy(bridge.at[pl.ds(2 * B + slot0, 64)], nh_v, sem)
            a4 = pltpu.async_copy(bridge.at[pl.ds(3 * B + slot0, 64)], nt_v, sem)
            a1.wait(); a2.wait(); a3.wait(); a4.wait()
            score_pass(h_v, t_v, c * 64, og)
            score_pass(nh_v, nt_v, c * 64, on)
            return carry

        lax.fori_loop(0, PW // 64, chunk_body, jnp.int32(0))
        pltpu.sync_copy(og, out_g.at[pl.ds(base, PW)])
        pltpu.sync_copy(on, out_n.at[pl.ds(base, PW)])

    return k2


def kernel(heads, tails, negative_heads, negative_tails, relations,
           entity_embeddings, relation_embeddings):
    tt = entity_embeddings.T
    relp = relation_embeddings.reshape(500, 128)
    bridge = _make_k1()(heads, tails, negative_heads, negative_tails, tt)
    return _make_k2()(bridge, relp, relations)
